# baseline (speedup 1.0000x reference)
import functools

import jax
import jax.numpy as jnp
from jax import lax
from jax.experimental import pallas as pl
from jax.experimental.pallas import tpu as pltpu

NUM_EXPERTS_K = 64
TOP_K_K = 8
HIDDEN_K = 2048
TOKENS_K = 32768
BLOCK_T = 1024


def _router_block(x_ref, wt_ref, logits_ref, topv_ref, topi_ref):
    x = x_ref[...]
    wt = wt_ref[...]
    logits = jnp.dot(x, wt, preferred_element_type=jnp.float32)
    logits_ref[...] = logits

    work = logits.T
    iota = lax.broadcasted_iota(
        jnp.int32, (NUM_EXPERTS_K, BLOCK_T), 0).astype(jnp.float32)
    vals = []
    idxs = []
    for _ in range(TOP_K_K):
        v = jnp.max(work, axis=0, keepdims=True)
        i = jnp.min(jnp.where(work == v, iota, float(NUM_EXPERTS_K)), axis=0,
                    keepdims=True)
        vals.append(v)
        idxs.append(i)
        work = jnp.where(iota == i, -jnp.inf, work)

    topl = jnp.concatenate(vals, axis=0)
    e = jnp.exp(topl - vals[0])
    topv_ref[...] = (e / jnp.sum(e, axis=0, keepdims=True)).T
    topi_ref[...] = jnp.concatenate(idxs, axis=0).T.astype(jnp.int32)


@jax.jit
def kernel(hidden_states, weight):
    wt = weight.T
    grid = (TOKENS_K // BLOCK_T,)
    logits, topv, topi = pl.pallas_call(
        _router_block,
        grid=grid,
        in_specs=[
            pl.BlockSpec((BLOCK_T, HIDDEN_K), lambda i: (i, 0)),
            pl.BlockSpec((HIDDEN_K, NUM_EXPERTS_K), lambda i: (0, 0)),
        ],
        out_specs=[
            pl.BlockSpec((BLOCK_T, NUM_EXPERTS_K), lambda i: (i, 0)),
            pl.BlockSpec((BLOCK_T, TOP_K_K), lambda i: (i, 0)),
            pl.BlockSpec((BLOCK_T, TOP_K_K), lambda i: (i, 0)),
        ],
        out_shape=[
            jax.ShapeDtypeStruct((TOKENS_K, NUM_EXPERTS_K), jnp.float32),
            jax.ShapeDtypeStruct((TOKENS_K, TOP_K_K), jnp.float32),
            jax.ShapeDtypeStruct((TOKENS_K, TOP_K_K), jnp.int32),
        ],
        compiler_params=pltpu.CompilerParams(
            dimension_semantics=("parallel",),
        ),
    )(hidden_states, wt)
    return logits, topv, topi


# speedup vs baseline: 1.0228x; 1.0228x over previous
import jax
import jax.numpy as jnp
from jax import lax
from jax.experimental import pallas as pl
from jax.experimental.pallas import tpu as pltpu

NUM_EXPERTS_K = 64
TOP_K_K = 8
HIDDEN_K = 2048
TOKENS_K = 32768
BLOCK_T = 2048


def _router_block(x_ref, wt_ref, logits_ref, topv_ref, topi_ref):
    x = x_ref[...]
    wt = wt_ref[...]
    logits = jnp.dot(x, wt, preferred_element_type=jnp.float32)
    logits_ref[...] = logits

    work = logits.T
    iota = lax.broadcasted_iota(
        jnp.int32, (NUM_EXPERTS_K, BLOCK_T), 0).astype(jnp.float32)
    vals = []
    idxs = []
    for _ in range(TOP_K_K):
        v = jnp.max(work, axis=0, keepdims=True)
        i = jnp.min(jnp.where(work == v, iota, float(NUM_EXPERTS_K)), axis=0,
                    keepdims=True)
        vals.append(v)
        idxs.append(i)
        work = jnp.where(iota == i, -jnp.inf, work)

    topl = jnp.concatenate(vals, axis=0)
    e = jnp.exp(topl - vals[0])
    topv_ref[...] = (e / jnp.sum(e, axis=0, keepdims=True)).T
    topi_ref[...] = jnp.concatenate(idxs, axis=0).T.astype(jnp.int32)


@jax.jit
def kernel(hidden_states, weight):
    wt = weight.T
    grid = (TOKENS_K // BLOCK_T,)
    logits, topv, topi = pl.pallas_call(
        _router_block,
        grid=grid,
        in_specs=[
            pl.BlockSpec((BLOCK_T, HIDDEN_K), lambda i: (i, 0)),
            pl.BlockSpec((HIDDEN_K, NUM_EXPERTS_K), lambda i: (0, 0)),
        ],
        out_specs=[
            pl.BlockSpec((BLOCK_T, NUM_EXPERTS_K), lambda i: (i, 0)),
            pl.BlockSpec((BLOCK_T, TOP_K_K), lambda i: (i, 0)),
            pl.BlockSpec((BLOCK_T, TOP_K_K), lambda i: (i, 0)),
        ],
        out_shape=[
            jax.ShapeDtypeStruct((TOKENS_K, NUM_EXPERTS_K), jnp.float32),
            jax.ShapeDtypeStruct((TOKENS_K, TOP_K_K), jnp.float32),
            jax.ShapeDtypeStruct((TOKENS_K, TOP_K_K), jnp.int32),
        ],
        compiler_params=pltpu.CompilerParams(
            dimension_semantics=("parallel",),
        ),
    )(hidden_states, wt)
    return logits, topv, topi
